# baseline (speedup 1.0000x reference)
import functools

import jax
import jax.numpy as jnp
from jax import lax
from jax.experimental import pallas as pl
from jax.experimental.pallas import tpu as pltpu
from jax.experimental.pallas import tpu_sc as plsc

D_MODEL = 64
SCALE = 8.0

NC = 2
NS = 16
NW = NC * NS
LANES = 16

BBLK = 128


def _make_kernel(n_b, n_t):
    assert n_b % (NW * BBLK) == 0 and n_b // BBLK == NW
    dt = D_MODEL // 8

    mesh = plsc.VectorSubcoreMesh(core_axis_name="c", subcore_axis_name="s")

    @functools.partial(
        pl.kernel,
        out_type=jax.ShapeDtypeStruct((n_t, dt, NW, 8, BBLK), jnp.float32),
        mesh=mesh,
        compiler_params=pltpu.CompilerParams(use_tc_tiling_on_sc=True),
        scratch_types=[
            pltpu.VMEM((n_t, BBLK), jnp.int32),
            pltpu.VMEM((D_MODEL, BBLK), jnp.float32),
            pltpu.VMEM((D_MODEL, BBLK), jnp.float32),
            pltpu.SemaphoreType.DMA,
            pltpu.SemaphoreType.DMA,
            pltpu.SemaphoreType.DMA,
            pltpu.SemaphoreType.DMA,
        ],
    )
    def emb_kernel(x_hbm, tab_hbm, out_hbm, idx_all, tp0, tp1,
                   gsem0, gsem1, wsem0, wsem1):
        wid = lax.axis_index("s") * NC + lax.axis_index("c")
        tp_bufs = (tp0, tp1)
        gsems = (gsem0, gsem1)
        wsems = (wsem0, wsem1)

        def fetch_chunk(t, p):
            tp_v, sem = tp_bufs[p], gsems[p]

            def g_body(g, carry):
                vec = idx_all[t, pl.ds(g * LANES, LANES)]
                for j in range(LANES):
                    pltpu.async_copy(
                        tab_hbm.at[vec[j]],
                        tp_v.at[:, g * LANES + j],
                        sem,
                    )
                return carry

            lax.fori_loop(0, BBLK // LANES, g_body, 0)

        def process(t, p):
            tp_v = tp_bufs[p]
            pltpu.make_async_copy(
                x_hbm.at[pl.ds(0, D_MODEL), 0], tp_v, gsems[p]
            ).wait()

            def scale_d(d, carry):
                for bg in range(BBLK // LANES):
                    sl = pl.ds(bg * LANES, LANES)
                    tp_v[d, sl] = tp_v[d, sl] * SCALE
                return carry

            lax.fori_loop(0, D_MODEL, scale_d, 0)

            for dti in range(dt):
                pltpu.async_copy(
                    tp_v.at[pl.ds(dti * 8, 8)],
                    out_hbm.at[t, dti, wid],
                    wsems[p],
                )

        def wait_write(p):
            pltpu.make_async_copy(
                x_hbm.at[pl.ds(0, D_MODEL), 0], tp_bufs[p], wsems[p]
            ).wait()

        def step(t, carry):
            p = lax.rem(t, 2)

            @pl.when(p == 0)
            def _():
                @pl.when(t + 1 < n_t)
                def _():
                    @pl.when(t >= 1)
                    def _():
                        wait_write(1)
                    fetch_chunk(t + 1, 1)
                process(t, 0)

            @pl.when(p == 1)
            def _():
                @pl.when(t + 1 < n_t)
                def _():
                    wait_write(0)
                    fetch_chunk(t + 1, 0)
                process(t, 1)

            return carry

        pltpu.sync_copy(x_hbm.at[:, wid], idx_all)
        fetch_chunk(0, 0)
        lax.fori_loop(0, n_t, step, 0)
        wait_write(0)
        wait_write(1)

    return emb_kernel


def kernel(x, table):
    n_b, n_t = x.shape
    xt = x.T.reshape(n_t, NW, BBLK).astype(jnp.int32)
    out5 = _make_kernel(n_b, n_t)(xt, table)
    return out5.transpose(2, 4, 0, 1, 3).reshape(n_b, n_t, D_MODEL)
